# baseline (speedup 1.0000x reference)
import functools

import jax
import jax.numpy as jnp
from jax import lax
from jax.experimental import pallas as pl
from jax.experimental.pallas import tpu as pltpu
from jax.experimental.pallas import tpu_sc as plsc

BATCH = 16384
EMB = 32
NC = 2
NS = 16
NW = NC * NS
B_PER_W = BATCH // NW


G = 4
NGRP = B_PER_W // G


def _group_offsets(idx_v, base, woff):
    window = idx_v[pl.ds(pl.multiple_of(base, 8), 16)]
    res = []
    for k in range(G):
        r = window[woff + k]
        t0 = (r // 128) * 128
        res.append((pl.multiple_of(t0, 128), r - t0))
    return res


def _issue_group(tab, idx_v, chunks, p, base, woff, sem):
    for k, (t0, _) in enumerate(_group_offsets(idx_v, base, woff)):
        pltpu.async_copy(tab.at[:, pl.ds(t0, 128)], chunks.at[p, k], sem)


def _wait_group(tab, idx_v, chunks, p, base, woff, sem):
    for k, (t0, _) in enumerate(_group_offsets(idx_v, base, woff)):
        pltpu.make_async_copy(tab.at[:, pl.ds(t0, 128)],
                              chunks.at[p, k], sem).wait()


def _extract_group(idx_v, chunks, p, base, woff, buf):
    c16 = lax.iota(jnp.int32, 16)
    for k, (_, lane) in enumerate(_group_offsets(idx_v, base, woff)):
        lanev = jnp.full((16,), lane, jnp.int32)
        colv = jnp.full((16,), base + woff + k, jnp.int32)
        chunk = chunks.at[p, k]
        v0 = plsc.load_gather(chunk, [c16, lanev])
        v1 = plsc.load_gather(chunk, [c16 + 16, lanev])
        plsc.store_scatter(buf, [c16, colv], v0)
        plsc.store_scatter(buf, [c16 + 16, colv], v1)


def _gather_one_table(tab, idx_v, chunks, buf, sem0, sem1):
    WOFF1 = G % 8
    WSH1 = 8 * (G // 8)
    _issue_group(tab, idx_v, chunks, 0, 0, 0, sem0)

    def step(it, carry):
        base = it * 2 * G
        _issue_group(tab, idx_v, chunks, 1, base + WSH1, WOFF1, sem1)
        _wait_group(tab, idx_v, chunks, 0, base, 0, sem0)
        _extract_group(idx_v, chunks, 0, base, 0, buf)
        _issue_group(tab, idx_v, chunks, 0, base + 2 * G, 0, sem0)
        _wait_group(tab, idx_v, chunks, 1, base + WSH1, WOFF1, sem1)
        _extract_group(idx_v, chunks, 1, base + WSH1, WOFF1, buf)
        return carry

    lax.fori_loop(0, NGRP // 2 - 1, step, 0)
    base = (NGRP - 2) * G
    _issue_group(tab, idx_v, chunks, 1, base + WSH1, WOFF1, sem1)
    _wait_group(tab, idx_v, chunks, 0, base, 0, sem0)
    _extract_group(idx_v, chunks, 0, base, 0, buf)
    _wait_group(tab, idx_v, chunks, 1, base + WSH1, WOFF1, sem1)
    _extract_group(idx_v, chunks, 1, base + WSH1, WOFF1, buf)


def _gather_body(uidx_hbm, iidx_hbm, utab_hbm, itab_hbm, u_out, i_out,
                 uidx_v, iidx_v, ubuf, ibuf, chunks, sem0, sem1):
    wid = lax.axis_index("s") * NC + lax.axis_index("c")
    base = wid * B_PER_W
    pltpu.sync_copy(uidx_hbm.at[pl.ds(base, B_PER_W)],
                    uidx_v.at[pl.ds(0, B_PER_W)])
    pltpu.sync_copy(iidx_hbm.at[pl.ds(base, B_PER_W)],
                    iidx_v.at[pl.ds(0, B_PER_W)])
    _gather_one_table(utab_hbm, uidx_v, chunks, ubuf, sem0, sem1)
    pltpu.sync_copy(ubuf, u_out.at[:, pl.ds(base, B_PER_W)])
    _gather_one_table(itab_hbm, iidx_v, chunks, ibuf, sem0, sem1)
    pltpu.sync_copy(ibuf, i_out.at[:, pl.ds(base, B_PER_W)])


@functools.lru_cache(maxsize=1)
def _make_gather():
    return pl.kernel(
        _gather_body,
        out_type=(
            jax.ShapeDtypeStruct((EMB, BATCH), jnp.float32),
            jax.ShapeDtypeStruct((EMB, BATCH), jnp.float32),
        ),
        mesh=plsc.VectorSubcoreMesh(
            core_axis_name="c", subcore_axis_name="s",
            num_cores=NC, num_subcores=NS),
        compiler_params=pltpu.CompilerParams(
            disable_bounds_checks=True, needs_layout_passes=False),
        scratch_types=[
            pltpu.VMEM((B_PER_W + 16,), jnp.int32),
            pltpu.VMEM((B_PER_W + 16,), jnp.int32),
            pltpu.VMEM((EMB, B_PER_W), jnp.float32),
            pltpu.VMEM((EMB, B_PER_W), jnp.float32),
            pltpu.VMEM((2, G, EMB, 128), jnp.float32),
            pltpu.SemaphoreType.DMA,
            pltpu.SemaphoreType.DMA,
        ],
    )


def _mlp_body(u_ref, i_ref, w1u_ref, w1i_ref, b1_ref, w2_ref, b2_ref,
              w3_ref, b3_ref, out_ref):
    cdim = (((0,), (0,)), ((), ()))
    h = lax.dot_general(u_ref[...], w1u_ref[...], cdim,
                        preferred_element_type=jnp.float32)
    h = h + lax.dot_general(i_ref[...], w1i_ref[...], cdim,
                            preferred_element_type=jnp.float32)
    h = jnp.maximum(h + b1_ref[...], 0.0)
    h = jnp.dot(h, w2_ref[...], preferred_element_type=jnp.float32)
    h = jnp.maximum(h + b2_ref[...], 0.0)
    logit = jnp.sum(h * w3_ref[...], axis=1, keepdims=True) + b3_ref[...]
    out_ref[...] = jax.nn.sigmoid(logit)


def _mlp(u_t, i_t, w1u, w1i, b1, w2, b2, w3, b3, block=2048):
    nblk = BATCH // block
    full = lambda shape: pl.BlockSpec(shape, lambda b: (0, 0))
    return pl.pallas_call(
        _mlp_body,
        grid=(nblk,),
        in_specs=[
            pl.BlockSpec((EMB, block), lambda b: (0, b)),
            pl.BlockSpec((EMB, block), lambda b: (0, b)),
            full((EMB, 128)),
            full((EMB, 128)),
            full((1, 128)),
            full((128, 64)),
            full((1, 64)),
            full((1, 64)),
            full((1, 1)),
        ],
        out_specs=pl.BlockSpec((block, 1), lambda b: (b, 0)),
        out_shape=jax.ShapeDtypeStruct((BATCH, 1), jnp.float32),
    )(u_t, i_t, w1u, w1i, b1, w2, b2, w3, b3)


def kernel(user_idx, item_idx, user_table, item_table, W1, b1, W2, b2, W3, b3):
    u_t, i_t = _make_gather()(user_idx, item_idx, user_table.T, item_table.T)
    w1u = W1[:, :EMB].T
    w1i = W1[:, EMB:].T
    out = _mlp(u_t, i_t, w1u, w1i, b1.reshape(1, 128), W2.T,
               b2.reshape(1, 64), W3.reshape(1, 64), b3.reshape(1, 1))
    return (out, i_t.T)


# speedup vs baseline: 1.0887x; 1.0887x over previous
import functools

import jax
import jax.numpy as jnp
from jax import lax
from jax.experimental import pallas as pl
from jax.experimental.pallas import tpu as pltpu
from jax.experimental.pallas import tpu_sc as plsc

BATCH = 16384
EMB = 32
NC = 2
NS = 16
NW = NC * NS
B_PER_W = BATCH // NW


G = 8
NGRP = B_PER_W // G


def _group_offsets(idx_v, base, woff):
    window = idx_v[pl.ds(pl.multiple_of(base, 8), 16)]
    res = []
    for k in range(G):
        r = window[woff + k]
        t0 = (r // 128) * 128
        res.append((pl.multiple_of(t0, 128), r - t0))
    return res


def _issue_group(tab, idx_v, chunks, p, base, woff, sem):
    for k, (t0, _) in enumerate(_group_offsets(idx_v, base, woff)):
        pltpu.async_copy(tab.at[:, pl.ds(t0, 128)], chunks.at[p, k], sem)


def _wait_group(tab, idx_v, chunks, p, base, woff, sem):
    for k, (t0, _) in enumerate(_group_offsets(idx_v, base, woff)):
        pltpu.make_async_copy(tab.at[:, pl.ds(t0, 128)],
                              chunks.at[p, k], sem).wait()


def _extract_group(idx_v, chunks, p, base, woff, buf):
    c16 = lax.iota(jnp.int32, 16)
    for k, (_, lane) in enumerate(_group_offsets(idx_v, base, woff)):
        lanev = jnp.full((16,), lane, jnp.int32)
        colv = jnp.full((16,), base + woff + k, jnp.int32)
        chunk = chunks.at[p, k]
        v0 = plsc.load_gather(chunk, [c16, lanev])
        v1 = plsc.load_gather(chunk, [c16 + 16, lanev])
        plsc.store_scatter(buf, [c16, colv], v0)
        plsc.store_scatter(buf, [c16 + 16, colv], v1)


def _gather_one_table(tab, idx_v, chunks, buf, sem0, sem1, sem2):
    sems = (sem0, sem1, sem2)
    _issue_group(tab, idx_v, chunks, 0, 0, 0, sems[0])
    _issue_group(tab, idx_v, chunks, 1, G, 0, sems[1])

    def step(it, carry):
        for j in range(3):
            g = 3 * it + j
            base = g * G
            _wait_group(tab, idx_v, chunks, j, base, 0, sems[j])
            _extract_group(idx_v, chunks, j, base, 0, buf)
            j2 = (j + 2) % 3

            @pl.when(g + 2 < NGRP)
            def _():
                _issue_group(tab, idx_v, chunks, j2, (g + 2) * G, 0,
                             sems[j2])
        return carry

    lax.fori_loop(0, (NGRP - 1) // 3, step, 0)
    g = NGRP - 1
    _wait_group(tab, idx_v, chunks, g % 3, g * G, 0, sems[g % 3])
    _extract_group(idx_v, chunks, g % 3, g * G, 0, buf)


def _gather_body(uidx_hbm, iidx_hbm, utab_hbm, itab_hbm, u_out, i_out,
                 uidx_v, iidx_v, buf, chunks, sem0, sem1, sem2):
    wid = lax.axis_index("s") * NC + lax.axis_index("c")
    base = wid * B_PER_W
    pltpu.sync_copy(uidx_hbm.at[pl.ds(base, B_PER_W)],
                    uidx_v.at[pl.ds(0, B_PER_W)])
    pltpu.sync_copy(iidx_hbm.at[pl.ds(base, B_PER_W)],
                    iidx_v.at[pl.ds(0, B_PER_W)])
    _gather_one_table(utab_hbm, uidx_v, chunks, buf, sem0, sem1, sem2)
    pltpu.sync_copy(buf, u_out.at[:, pl.ds(base, B_PER_W)])
    _gather_one_table(itab_hbm, iidx_v, chunks, buf, sem0, sem1, sem2)
    pltpu.sync_copy(buf, i_out.at[:, pl.ds(base, B_PER_W)])


@functools.lru_cache(maxsize=1)
def _make_gather():
    return pl.kernel(
        _gather_body,
        out_type=(
            jax.ShapeDtypeStruct((EMB, BATCH), jnp.float32),
            jax.ShapeDtypeStruct((EMB, BATCH), jnp.float32),
        ),
        mesh=plsc.VectorSubcoreMesh(
            core_axis_name="c", subcore_axis_name="s",
            num_cores=NC, num_subcores=NS),
        compiler_params=pltpu.CompilerParams(
            disable_bounds_checks=True, needs_layout_passes=False),
        scratch_types=[
            pltpu.VMEM((B_PER_W + 16,), jnp.int32),
            pltpu.VMEM((B_PER_W + 16,), jnp.int32),
            pltpu.VMEM((EMB, B_PER_W), jnp.float32),
            pltpu.VMEM((3, G, EMB, 128), jnp.float32),
            pltpu.SemaphoreType.DMA,
            pltpu.SemaphoreType.DMA,
            pltpu.SemaphoreType.DMA,
        ],
    )


def _mlp_body(u_ref, i_ref, w1u_ref, w1i_ref, b1_ref, w2_ref, b2_ref,
              w3_ref, b3_ref, out_ref):
    cdim = (((0,), (0,)), ((), ()))
    h = lax.dot_general(u_ref[...], w1u_ref[...], cdim,
                        preferred_element_type=jnp.float32)
    h = h + lax.dot_general(i_ref[...], w1i_ref[...], cdim,
                            preferred_element_type=jnp.float32)
    h = jnp.maximum(h + b1_ref[...], 0.0)
    h = jnp.dot(h, w2_ref[...], preferred_element_type=jnp.float32)
    h = jnp.maximum(h + b2_ref[...], 0.0)
    logit = jnp.sum(h * w3_ref[...], axis=1, keepdims=True) + b3_ref[...]
    out_ref[...] = jax.nn.sigmoid(logit)


def _mlp(u_t, i_t, w1u, w1i, b1, w2, b2, w3, b3, block=2048):
    nblk = BATCH // block
    full = lambda shape: pl.BlockSpec(shape, lambda b: (0, 0))
    return pl.pallas_call(
        _mlp_body,
        grid=(nblk,),
        in_specs=[
            pl.BlockSpec((EMB, block), lambda b: (0, b)),
            pl.BlockSpec((EMB, block), lambda b: (0, b)),
            full((EMB, 128)),
            full((EMB, 128)),
            full((1, 128)),
            full((128, 64)),
            full((1, 64)),
            full((1, 64)),
            full((1, 1)),
        ],
        out_specs=pl.BlockSpec((block, 1), lambda b: (b, 0)),
        out_shape=jax.ShapeDtypeStruct((BATCH, 1), jnp.float32),
    )(u_t, i_t, w1u, w1i, b1, w2, b2, w3, b3)


def kernel(user_idx, item_idx, user_table, item_table, W1, b1, W2, b2, W3, b3):
    u_t, i_t = _make_gather()(user_idx, item_idx, user_table.T, item_table.T)
    w1u = W1[:, :EMB].T
    w1i = W1[:, EMB:].T
    out = _mlp(u_t, i_t, w1u, w1i, b1.reshape(1, 128), W2.T,
               b2.reshape(1, 64), W3.reshape(1, 64), b3.reshape(1, 1))
    return (out, i_t.T)


# speedup vs baseline: 1.0929x; 1.0039x over previous
import functools

import jax
import jax.numpy as jnp
from jax import lax
from jax.experimental import pallas as pl
from jax.experimental.pallas import tpu as pltpu
from jax.experimental.pallas import tpu_sc as plsc

BATCH = 16384
EMB = 32
NC = 2
NS = 16
NW = NC * NS
B_PER_W = BATCH // NW


G = 8
NGRP = B_PER_W // G


def _group_offsets(idx_v, base, woff):
    window = idx_v[pl.ds(pl.multiple_of(base, 8), 16)]
    res = []
    for k in range(G):
        r = window[woff + k]
        t0 = (r // 128) * 128
        res.append((pl.multiple_of(t0, 128), r - t0))
    return res


def _issue_group(tab, idx_v, chunks, p, base, woff, sem):
    for k, (t0, _) in enumerate(_group_offsets(idx_v, base, woff)):
        pltpu.async_copy(tab.at[:, pl.ds(t0, 128)], chunks.at[p, k], sem)


def _wait_group(tab, idx_v, chunks, p, base, woff, sem):
    for k, (t0, _) in enumerate(_group_offsets(idx_v, base, woff)):
        pltpu.make_async_copy(tab.at[:, pl.ds(t0, 128)],
                              chunks.at[p, k], sem).wait()


def _extract_group(idx_v, chunks, p, base, woff, buf):
    c16 = lax.iota(jnp.int32, 16)
    for k, (_, lane) in enumerate(_group_offsets(idx_v, base, woff)):
        lanev = jnp.full((16,), lane, jnp.int32)
        colv = jnp.full((16,), base + woff + k, jnp.int32)
        chunk = chunks.at[p, k]
        v0 = plsc.load_gather(chunk, [c16, lanev])
        v1 = plsc.load_gather(chunk, [c16 + 16, lanev])
        plsc.store_scatter(buf, [c16, colv], v0)
        plsc.store_scatter(buf, [c16 + 16, colv], v1)


def _gather_one_table(tab, idx_v, chunks, buf, sem0, sem1, sem2):
    sems = (sem0, sem1, sem2)
    _issue_group(tab, idx_v, chunks, 0, 0, 0, sems[0])
    _issue_group(tab, idx_v, chunks, 1, G, 0, sems[1])

    def step(it, carry):
        for j in range(3):
            g = 3 * it + j
            base = g * G
            _wait_group(tab, idx_v, chunks, j, base, 0, sems[j])
            _extract_group(idx_v, chunks, j, base, 0, buf)
            j2 = (j + 2) % 3

            @pl.when(g + 2 < NGRP)
            def _():
                _issue_group(tab, idx_v, chunks, j2, (g + 2) * G, 0,
                             sems[j2])
        return carry

    lax.fori_loop(0, (NGRP - 1) // 3, step, 0)
    g = NGRP - 1
    _wait_group(tab, idx_v, chunks, g % 3, g * G, 0, sems[g % 3])
    _extract_group(idx_v, chunks, g % 3, g * G, 0, buf)


def _gather_body(uidx_hbm, iidx_hbm, utab_hbm, itab_hbm, u_out, i_out,
                 uidx_v, iidx_v, buf, chunks, sem0, sem1, sem2):
    wid = lax.axis_index("s") * NC + lax.axis_index("c")
    base = wid * B_PER_W
    pltpu.sync_copy(uidx_hbm.at[pl.ds(base, B_PER_W)],
                    uidx_v.at[pl.ds(0, B_PER_W)])
    pltpu.sync_copy(iidx_hbm.at[pl.ds(base, B_PER_W)],
                    iidx_v.at[pl.ds(0, B_PER_W)])
    _gather_one_table(utab_hbm, uidx_v, chunks, buf, sem0, sem1, sem2)
    pltpu.sync_copy(buf, u_out.at[:, pl.ds(base, B_PER_W)])
    _gather_one_table(itab_hbm, iidx_v, chunks, buf, sem0, sem1, sem2)
    pltpu.sync_copy(buf, i_out.at[:, pl.ds(base, B_PER_W)])


@functools.lru_cache(maxsize=1)
def _make_gather():
    return pl.kernel(
        _gather_body,
        out_type=(
            jax.ShapeDtypeStruct((EMB, BATCH), jnp.float32),
            jax.ShapeDtypeStruct((EMB, BATCH), jnp.float32),
        ),
        mesh=plsc.VectorSubcoreMesh(
            core_axis_name="c", subcore_axis_name="s",
            num_cores=NC, num_subcores=NS),
        compiler_params=pltpu.CompilerParams(
            disable_bounds_checks=True, needs_layout_passes=False),
        scratch_types=[
            pltpu.VMEM((B_PER_W + 16,), jnp.int32),
            pltpu.VMEM((B_PER_W + 16,), jnp.int32),
            pltpu.VMEM((EMB, B_PER_W), jnp.float32),
            pltpu.VMEM((3, G, EMB, 128), jnp.float32),
            pltpu.SemaphoreType.DMA,
            pltpu.SemaphoreType.DMA,
            pltpu.SemaphoreType.DMA,
        ],
    )


def _mlp_body(u_ref, i_ref, w1u_ref, w1i_ref, b1_ref, w2_ref, b2_ref,
              w3_ref, b3_ref, out_ref):
    cdim = (((0,), (0,)), ((), ()))
    h = lax.dot_general(u_ref[...], w1u_ref[...], cdim,
                        preferred_element_type=jnp.float32)
    h = h + lax.dot_general(i_ref[...], w1i_ref[...], cdim,
                            preferred_element_type=jnp.float32)
    h = jnp.maximum(h + b1_ref[...], 0.0)
    h = jnp.dot(h, w2_ref[...], preferred_element_type=jnp.float32)
    h = jnp.maximum(h + b2_ref[...], 0.0)
    logit = jnp.sum(h * w3_ref[...], axis=1, keepdims=True) + b3_ref[...]
    out_ref[...] = jax.nn.sigmoid(logit)


def _mlp(u_t, i_t, w1u, w1i, b1, w2, b2, w3, b3, block=4096):
    nblk = BATCH // block
    full = lambda shape: pl.BlockSpec(shape, lambda b: (0, 0))
    return pl.pallas_call(
        _mlp_body,
        grid=(nblk,),
        in_specs=[
            pl.BlockSpec((EMB, block), lambda b: (0, b)),
            pl.BlockSpec((EMB, block), lambda b: (0, b)),
            full((EMB, 128)),
            full((EMB, 128)),
            full((1, 128)),
            full((128, 64)),
            full((1, 64)),
            full((1, 64)),
            full((1, 1)),
        ],
        out_specs=pl.BlockSpec((block, 1), lambda b: (b, 0)),
        out_shape=jax.ShapeDtypeStruct((BATCH, 1), jnp.float32),
    )(u_t, i_t, w1u, w1i, b1, w2, b2, w3, b3)


def kernel(user_idx, item_idx, user_table, item_table, W1, b1, W2, b2, W3, b3):
    u_t, i_t = _make_gather()(user_idx, item_idx, user_table.T, item_table.T)
    w1u = W1[:, :EMB].T
    w1i = W1[:, EMB:].T
    out = _mlp(u_t, i_t, w1u, w1i, b1.reshape(1, 128), W2.T,
               b2.reshape(1, 64), W3.reshape(1, 64), b3.reshape(1, 1))
    return (out, i_t.T)


# speedup vs baseline: 1.0960x; 1.0028x over previous
import functools

import jax
import jax.numpy as jnp
from jax import lax
from jax.experimental import pallas as pl
from jax.experimental.pallas import tpu as pltpu
from jax.experimental.pallas import tpu_sc as plsc

BATCH = 16384
EMB = 32
NC = 2
NS = 16
NW = NC * NS
B_PER_W = BATCH // NW


G = 8
NGRP = B_PER_W // G


def _group_offsets(idx_v, base, woff):
    window = idx_v[pl.ds(pl.multiple_of(base, 8), 16)]
    res = []
    for k in range(G):
        r = window[woff + k]
        t0 = (r // 128) * 128
        res.append((pl.multiple_of(t0, 128), r - t0))
    return res


def _issue_group(tab, idx_v, chunks, p, base, woff, sem):
    for k, (t0, _) in enumerate(_group_offsets(idx_v, base, woff)):
        pltpu.async_copy(tab.at[:, pl.ds(t0, 128)], chunks.at[p, k], sem)


def _wait_group(tab, idx_v, chunks, p, base, woff, sem):
    for k, (t0, _) in enumerate(_group_offsets(idx_v, base, woff)):
        pltpu.make_async_copy(tab.at[:, pl.ds(t0, 128)],
                              chunks.at[p, k], sem).wait()


def _extract_group(idx_v, chunks, p, base, woff, buf):
    c16 = lax.iota(jnp.int32, 16)
    for k, (_, lane) in enumerate(_group_offsets(idx_v, base, woff)):
        lanev = jnp.full((16,), lane, jnp.int32)
        colv = jnp.full((16,), base + woff + k, jnp.int32)
        chunk = chunks.at[p, k]
        v0 = plsc.load_gather(chunk, [c16, lanev])
        v1 = plsc.load_gather(chunk, [c16 + 16, lanev])
        plsc.store_scatter(buf, [c16, colv], v0)
        plsc.store_scatter(buf, [c16 + 16, colv], v1)


def _gather_one_table(tab, idx_v, chunks, buf, sem0, sem1, sem2):
    sems = (sem0, sem1, sem2)
    _issue_group(tab, idx_v, chunks, 0, 0, 0, sems[0])
    _issue_group(tab, idx_v, chunks, 1, G, 0, sems[1])

    def step(it, carry):
        for j in range(3):
            g = 3 * it + j
            base = g * G
            _wait_group(tab, idx_v, chunks, j, base, 0, sems[j])
            _extract_group(idx_v, chunks, j, base, 0, buf)
            j2 = (j + 2) % 3

            @pl.when(g + 2 < NGRP)
            def _():
                _issue_group(tab, idx_v, chunks, j2, (g + 2) * G, 0,
                             sems[j2])
        return carry

    lax.fori_loop(0, (NGRP - 1) // 3, step, 0)
    g = NGRP - 1
    _wait_group(tab, idx_v, chunks, g % 3, g * G, 0, sems[g % 3])
    _extract_group(idx_v, chunks, g % 3, g * G, 0, buf)


def _gather_body(uidx_hbm, iidx_hbm, utab_hbm, itab_hbm, u_out, i_out,
                 uidx_v, iidx_v, buf, chunks, sem0, sem1, sem2):
    wid = lax.axis_index("s") * NC + lax.axis_index("c")
    base = wid * B_PER_W
    pltpu.sync_copy(uidx_hbm.at[pl.ds(base, B_PER_W)],
                    uidx_v.at[pl.ds(0, B_PER_W)])
    pltpu.sync_copy(iidx_hbm.at[pl.ds(base, B_PER_W)],
                    iidx_v.at[pl.ds(0, B_PER_W)])
    _gather_one_table(utab_hbm, uidx_v, chunks, buf, sem0, sem1, sem2)
    pltpu.sync_copy(buf, u_out.at[:, pl.ds(base, B_PER_W)])
    _gather_one_table(itab_hbm, iidx_v, chunks, buf, sem0, sem1, sem2)
    pltpu.sync_copy(buf, i_out.at[:, pl.ds(base, B_PER_W)])


@functools.lru_cache(maxsize=1)
def _make_gather():
    return pl.kernel(
        _gather_body,
        out_type=(
            jax.ShapeDtypeStruct((EMB, BATCH), jnp.float32),
            jax.ShapeDtypeStruct((EMB, BATCH), jnp.float32),
        ),
        mesh=plsc.VectorSubcoreMesh(
            core_axis_name="c", subcore_axis_name="s",
            num_cores=NC, num_subcores=NS),
        compiler_params=pltpu.CompilerParams(
            disable_bounds_checks=True, needs_layout_passes=False),
        scratch_types=[
            pltpu.VMEM((B_PER_W + 16,), jnp.int32),
            pltpu.VMEM((B_PER_W + 16,), jnp.int32),
            pltpu.VMEM((EMB, B_PER_W), jnp.float32),
            pltpu.VMEM((3, G, EMB, 128), jnp.float32),
            pltpu.SemaphoreType.DMA,
            pltpu.SemaphoreType.DMA,
            pltpu.SemaphoreType.DMA,
        ],
    )


def _mlp_body(u_ref, i_ref, w1u_ref, w1i_ref, b1_ref, w2_ref, b2_ref,
              w3_ref, b3_ref, out_ref):
    cdim = (((0,), (0,)), ((), ()))
    h = lax.dot_general(u_ref[...], w1u_ref[...], cdim,
                        preferred_element_type=jnp.float32)
    h = h + lax.dot_general(i_ref[...], w1i_ref[...], cdim,
                            preferred_element_type=jnp.float32)
    h = jnp.maximum(h + b1_ref[...], 0.0)
    h = jnp.dot(h, w2_ref[...], preferred_element_type=jnp.float32)
    h = jnp.maximum(h + b2_ref[...], 0.0)
    logit = jnp.sum(h * w3_ref[...], axis=1, keepdims=True) + b3_ref[...]
    out_ref[...] = jax.nn.sigmoid(logit)


def _mlp(u_t, i_t, w1u, w1i, b1, w2, b2, w3, b3, block=8192):
    nblk = BATCH // block
    full = lambda shape: pl.BlockSpec(shape, lambda b: (0, 0))
    return pl.pallas_call(
        _mlp_body,
        grid=(nblk,),
        in_specs=[
            pl.BlockSpec((EMB, block), lambda b: (0, b)),
            pl.BlockSpec((EMB, block), lambda b: (0, b)),
            full((EMB, 128)),
            full((EMB, 128)),
            full((1, 128)),
            full((128, 64)),
            full((1, 64)),
            full((1, 64)),
            full((1, 1)),
        ],
        out_specs=pl.BlockSpec((block, 1), lambda b: (b, 0)),
        out_shape=jax.ShapeDtypeStruct((BATCH, 1), jnp.float32),
    )(u_t, i_t, w1u, w1i, b1, w2, b2, w3, b3)


def kernel(user_idx, item_idx, user_table, item_table, W1, b1, W2, b2, W3, b3):
    u_t, i_t = _make_gather()(user_idx, item_idx, user_table.T, item_table.T)
    w1u = W1[:, :EMB].T
    w1i = W1[:, EMB:].T
    out = _mlp(u_t, i_t, w1u, w1i, b1.reshape(1, 128), W2.T,
               b2.reshape(1, 64), W3.reshape(1, 64), b3.reshape(1, 1))
    return (out, i_t.T)
